# baseline (speedup 1.0000x reference)
import functools

import jax
import jax.numpy as jnp
from jax import lax
from jax.experimental import pallas as pl
from jax.experimental.pallas import tpu as pltpu
from jax.experimental.pallas import tpu_sc as plsc

NUM_EMBEDDINGS = 1000000
EMBEDDING_DIM = 64
BATCH = 4096 * 200

NUM_CORES = 2
NUM_SUBCORES = 16
NUM_WORKERS = NUM_CORES * NUM_SUBCORES

GROUP = 256
TOK_PER_WORKER = BATCH // NUM_WORKERS
NGRP = TOK_PER_WORKER // GROUP

_mesh = plsc.VectorSubcoreMesh(core_axis_name="c", subcore_axis_name="s")


@functools.partial(
    pl.kernel,
    out_type=jax.ShapeDtypeStruct((BATCH, EMBEDDING_DIM), jnp.float32),
    mesh=_mesh,
    compiler_params=pltpu.CompilerParams(use_tc_tiling_on_sc=False),
    scratch_types=[
        pltpu.VMEM((TOK_PER_WORKER,), jnp.int32),
        pltpu.VMEM_SHARED((NUM_SUBCORES, 2, GROUP), jnp.int32),
        pltpu.SMEM((2, GROUP), jnp.int32),
        pltpu.SemaphoreType.DMA,
    ],
)
def _embed_sc(table_hbm, idx_hbm, out_hbm, idx_v, idx_sh, idx_s, sem_g):
    sid = lax.axis_index("s")
    wid = sid * NUM_CORES + lax.axis_index("c")
    tok_base = wid * TOK_PER_WORKER
    pltpu.sync_copy(idx_hbm.at[pl.ds(tok_base, TOK_PER_WORKER)], idx_v)

    def stage_idx(grp, buf):
        pltpu.sync_copy(idx_v.at[pl.ds(grp * GROUP, GROUP)], idx_sh.at[sid, buf])
        pltpu.sync_copy(idx_sh.at[sid, buf], idx_s.at[buf])

    def fire_copies(grp, buf):
        @pl.loop(0, GROUP, unroll=8)
        def _row(t):
            r = idx_s[buf, t]
            pltpu.async_copy(
                table_hbm.at[pl.ds(r, 1)],
                out_hbm.at[pl.ds(tok_base + grp * GROUP + t, 1)],
                sem_g,
            )

    stage_idx(0, 0)
    fire_copies(0, 0)

    @pl.loop(0, NGRP)
    def _group(g):
        buf = lax.rem(g, 2)

        @pl.when(g + 1 < NGRP)
        def _():
            stage_idx(g + 1, 1 - buf)
            fire_copies(g + 1, 1 - buf)

        pltpu.make_async_copy(
            table_hbm.at[pl.ds(0, GROUP)], out_hbm.at[pl.ds(0, GROUP)], sem_g
        ).wait()


def kernel(token_ids, weight):
    idx = token_ids.astype(jnp.int32).reshape(BATCH)
    out = _embed_sc(weight, idx)
    return out.reshape(token_ids.shape[0], token_ids.shape[1], EMBEDDING_DIM)


# speedup vs baseline: 5.9704x; 5.9704x over previous
import functools

import jax
import jax.numpy as jnp
from jax import lax
from jax.experimental import pallas as pl
from jax.experimental.pallas import tpu as pltpu
from jax.experimental.pallas import tpu_sc as plsc

NUM_EMBEDDINGS = 1000000
EMBEDDING_DIM = 64
BATCH = 4096 * 200

NUM_CORES = 2
NUM_SUBCORES = 16
NUM_WORKERS = NUM_CORES * NUM_SUBCORES

GROUP = 512
ROWS_PER_WORKER = BATCH // NUM_WORKERS
NGRP = ROWS_PER_WORKER // GROUP
NG_TOTAL = BATCH // GROUP
NBUF = 3

_mesh = plsc.VectorSubcoreMesh(core_axis_name="c", subcore_axis_name="s")


@functools.partial(
    pl.kernel,
    out_type=jax.ShapeDtypeStruct((BATCH, EMBEDDING_DIM), jnp.float32),
    mesh=_mesh,
    compiler_params=pltpu.CompilerParams(use_tc_tiling_on_sc=False),
    scratch_types=[
        pltpu.VMEM((NGRP, GROUP), jnp.int32),
        pltpu.VMEM((NBUF, GROUP, EMBEDDING_DIM), jnp.float32),
        pltpu.SemaphoreType.DMA,
        pltpu.SemaphoreType.DMA,
    ],
)
def _embed_sc(table_hbm, idx_hbm, out_hbm, idx_v, rows_v, sem_g, sem_s):
    wid = lax.axis_index("s") * NUM_CORES + lax.axis_index("c")
    base = wid * ROWS_PER_WORKER
    pltpu.sync_copy(idx_hbm.at[pl.ds(wid * NGRP, NGRP)], idx_v)

    def fire_gather(grp, buf):
        pltpu.async_copy(table_hbm.at[idx_v.at[grp]], rows_v.at[buf], sem_g)

    def drain_gather(buf):
        pltpu.make_async_copy(
            table_hbm.at[idx_v.at[0]], rows_v.at[buf], sem_g
        ).wait()

    fire_gather(0, 0)
    fire_gather(1, 1)

    @pl.loop(0, NGRP)
    def _group(g):
        buf = lax.rem(g, NBUF)
        drain_gather(buf)
        pltpu.async_copy(
            rows_v.at[buf], out_hbm.at[pl.ds(base + g * GROUP, GROUP)], sem_s
        )

        @pl.when(g + 2 < NGRP)
        def _():
            @pl.when(g >= 1)
            def _():
                pltpu.make_async_copy(
                    rows_v.at[buf], out_hbm.at[pl.ds(0, GROUP)], sem_s
                ).wait()

            fire_gather(g + 2, lax.rem(g + 2, NBUF))

    for _ in range(NBUF):
        pltpu.make_async_copy(
            rows_v.at[0], out_hbm.at[pl.ds(0, GROUP)], sem_s
        ).wait()


def kernel(token_ids, weight):
    idx = token_ids.astype(jnp.int32).reshape(NG_TOTAL, GROUP)
    out = _embed_sc(weight, idx)
    return out.reshape(token_ids.shape[0], token_ids.shape[1], EMBEDDING_DIM)
